# baseline (speedup 1.0000x reference)
import functools
import math

import jax
import jax.numpy as jnp
from jax import lax
from jax.experimental import pallas as pl
from jax.experimental.pallas import tpu as pltpu
from jax.experimental.pallas import tpu_sc as plsc

_N = 20000
_C = 81
_CF = 80
_K = 200
_DETS = 100
_IMG_W = 1333.0
_IMG_H = 800.0
_SCORE_T = 0.05
_NMS_T = 0.5
_CLIP = math.log(1000.0 / 16.0)

_NW = 32
_PW = 512
_A_BLK = 2000


def _softmax_body(logit_ref, out_ref):
    x = logit_ref[...]
    m = jnp.max(x, axis=1, keepdims=True)
    e = jnp.exp(x - m)
    s = jnp.sum(e, axis=1, keepdims=True)
    p = e / s
    fg = p[:, 1:]
    masked = jnp.where(fg > _SCORE_T, fg, -1.0)
    out_ref[...] = masked.T


def _masked_scores(class_logits):
    return pl.pallas_call(
        _softmax_body,
        out_shape=jax.ShapeDtypeStruct((_CF, _N), jnp.float32),
    )(class_logits)


_B05 = 1028443341
_B1 = 1065353216
_BN1 = -1082130432


def _bisect_body(sc_ref, thr_ref):
    kb = lax.bitcast_convert_type(sc_ref[...], jnp.int32)
    c05 = jnp.sum((kb > _B05).astype(jnp.int32), axis=1, keepdims=True)
    lo0 = jnp.full((_CF, 1), _B05, jnp.int32)
    hi0 = jnp.full((_CF, 1), _B1, jnp.int32)

    def body(t, carry):
        lo, hi = carry
        mid = (lo + hi) >> 1
        cnt = jnp.sum((kb > mid).astype(jnp.int32), axis=1, keepdims=True)
        small = cnt < _K
        return (jnp.where(small, lo, mid), jnp.where(small, mid, hi))

    lo, hi = lax.fori_loop(0, 26, body, (lo0, hi0))
    thr_bits = jnp.where(c05 >= _K, hi, jnp.int32(_BN1))
    thr_ref[...] = lax.bitcast_convert_type(thr_bits, jnp.float32)


def _bisect(masked_t):
    return pl.pallas_call(
        _bisect_body,
        out_shape=jax.ShapeDtypeStruct((_CF, 1), jnp.float32),
    )(masked_t)


_CAP = 224
_SCAN = _N // 16


def _compact_body(sc_hbm, thr_hbm, cs_hbm, ci_hbm,
                  row_v, thr_sv, gs_v, gi_v, ts_v, ti_v):
    ci_ax = lax.axis_index("c")
    si_ax = lax.axis_index("s")
    w = si_ax * 2 + ci_ax
    pltpu.sync_copy(thr_hbm, thr_sv)
    iota16 = lax.iota(jnp.int32, 16)
    nk = jnp.where(w < 16, 3, 2)
    base_c = jnp.where(w < 16, w * 3, 48 + (w - 16) * 2)
    for k in range(3):
        @pl.when(k < nk)
        def _():
            c = base_c + k
            pltpu.sync_copy(sc_hbm.at[c], row_v)
            thrv = thr_sv[pl.ds(c * 16, 16)]
            row_r = row_v.at[0]
            gs_r = gs_v.at[0]
            gi_r = gi_v.at[0]
            ts_r = ts_v.at[0]
            ti_r = ti_v.at[0]
            for j in range(_CAP // 16):
                gs_r[pl.ds(j * 16, 16)] = jnp.full((16,), -2.0)
                ts_r[pl.ds(j * 16, 16)] = jnp.full((16,), -2.0)

            def body(i, carry):
                cg, ct = carry
                i16 = i * 16
                v = row_r[pl.ds(i16, 16)]
                idxv = i16 + iota16
                mg = v > thrv
                plsc.store_compressed(gs_r.at[pl.ds(cg, 16)], v, mask=mg)
                plsc.store_compressed(gi_r.at[pl.ds(cg, 16)], idxv, mask=mg)
                cg = cg + jnp.max(plsc.all_reduce_population_count(mg))
                mt = v == thrv
                pos = plsc.cumsum(mt.astype(jnp.int32))
                sel = mt & ((pos + ct) <= _K)
                plsc.store_compressed(ts_r.at[pl.ds(ct, 16)], v, mask=sel)
                plsc.store_compressed(ti_r.at[pl.ds(ct, 16)], idxv, mask=sel)
                ct = ct + jnp.max(plsc.all_reduce_population_count(sel))
                return (cg, ct)

            lax.fori_loop(0, _SCAN, body, (jnp.int32(0), jnp.int32(0)))
            pltpu.sync_copy(gs_v, cs_hbm.at[c, 0])
            pltpu.sync_copy(ts_v, cs_hbm.at[c, 1])
            pltpu.sync_copy(gi_v, ci_hbm.at[c, 0])
            pltpu.sync_copy(ti_v, ci_hbm.at[c, 1])


def _compact(masked_sc, thr):
    mesh = plsc.VectorSubcoreMesh(core_axis_name="c", subcore_axis_name="s")
    fn = functools.partial(
        pl.kernel,
        mesh=mesh,
        out_type=[
            jax.ShapeDtypeStruct((_CF, 2, 1, _CAP), jnp.float32),
            jax.ShapeDtypeStruct((_CF, 2, 1, _CAP), jnp.int32),
        ],
        scratch_types=[
            pltpu.VMEM((1, _N), jnp.float32),
            pltpu.VMEM((_CF * 16,), jnp.float32),
            pltpu.VMEM((1, _CAP), jnp.float32),
            pltpu.VMEM((1, _CAP), jnp.int32),
            pltpu.VMEM((1, _CAP), jnp.float32),
            pltpu.VMEM((1, _CAP), jnp.int32),
        ],
    )(_compact_body)
    return fn(masked_sc, thr)


def _gather_body(reg_hbm, prop_hbm, ridx_hbm, pidx_hbm, oreg_hbm, oprop_hbm,
                 idxr_v, idxp_v, regrows_v, proprows_v, sem):
    c = lax.axis_index("c")
    s = lax.axis_index("s")
    w = s * 2 + c
    pltpu.sync_copy(ridx_hbm.at[w], idxr_v)
    pltpu.sync_copy(pidx_hbm.at[w], idxp_v)
    copies = []
    for ch in range(4):
        for j in range(_PW // 128):
            cp = pltpu.make_async_copy(
                reg_hbm.at[idxr_v.at[ch, j]],
                regrows_v.at[ch, pl.ds(j * 128, 128)], sem)
            cp.start()
            copies.append(cp)
            cp = pltpu.make_async_copy(
                prop_hbm.at[idxp_v.at[ch, j]],
                proprows_v.at[ch, pl.ds(j * 128, 128)], sem)
            cp.start()
            copies.append(cp)
    for cp in copies:
        cp.wait()
    pltpu.sync_copy(regrows_v, oreg_hbm.at[w])
    pltpu.sync_copy(proprows_v, oprop_hbm.at[w])


def _gather_candidates(reg_flat, props, ridx, pidx):
    mesh = plsc.VectorSubcoreMesh(core_axis_name="c", subcore_axis_name="s")
    fn = functools.partial(
        pl.kernel,
        mesh=mesh,
        out_type=[
            jax.ShapeDtypeStruct((_NW, 4, _PW), jnp.float32),
            jax.ShapeDtypeStruct((_NW, 4, _PW), jnp.float32),
        ],
        scratch_types=[
            pltpu.VMEM((4, _PW // 128, 128), jnp.int32),
            pltpu.VMEM((4, _PW // 128, 128), jnp.int32),
            pltpu.VMEM((4, _PW), jnp.float32),
            pltpu.VMEM((4, _PW), jnp.float32),
            pltpu.SemaphoreType.DMA,
        ],
    )(_gather_body)
    return fn(reg_flat, props, ridx, pidx)


def _nms_body(sc_ref, reg_ref, prop_ref, outs_ref, outb_ref, area_ref, keep_ref):
    px1 = prop_ref[0]
    py1 = prop_ref[1]
    px2 = prop_ref[2]
    py2 = prop_ref[3]
    widths = px2 - px1 + 1.0
    heights = py2 - py1 + 1.0
    ctr_x = px1 + 0.5 * widths
    ctr_y = py1 + 0.5 * heights
    dx = reg_ref[0] / 10.0
    dy = reg_ref[1] / 10.0
    dw = jnp.minimum(reg_ref[2] / 5.0, _CLIP)
    dh = jnp.minimum(reg_ref[3] / 5.0, _CLIP)
    pred_ctr_x = dx * widths + ctr_x
    pred_ctr_y = dy * heights + ctr_y
    pred_w = jnp.exp(dw) * widths
    pred_h = jnp.exp(dh) * heights
    x1 = jnp.clip(pred_ctr_x - 0.5 * pred_w, 0.0, _IMG_W - 1.0)
    y1 = jnp.clip(pred_ctr_y - 0.5 * pred_h, 0.0, _IMG_H - 1.0)
    x2 = jnp.clip(pred_ctr_x + 0.5 * pred_w - 1.0, 0.0, _IMG_W - 1.0)
    y2 = jnp.clip(pred_ctr_y + 0.5 * pred_h - 1.0, 0.0, _IMG_H - 1.0)
    outb_ref[0] = x1
    outb_ref[1] = y1
    outb_ref[2] = x2
    outb_ref[3] = y2
    area_ref[...] = (x2 - x1 + 1.0) * (y2 - y1 + 1.0)
    area = area_ref[...]
    sc = sc_ref[...]
    keep_ref[...] = jnp.where(sc > _SCORE_T, 1.0, 0.0)
    row = lax.broadcasted_iota(jnp.int32, (_K, _CF), 0)

    def body(i, carry):
        a1 = outb_ref[0, pl.ds(i, 1), :]
        b1 = outb_ref[1, pl.ds(i, 1), :]
        a2 = outb_ref[2, pl.ds(i, 1), :]
        b2 = outb_ref[3, pl.ds(i, 1), :]
        ai = area_ref[pl.ds(i, 1), :]
        ltx = jnp.maximum(x1, a1)
        lty = jnp.maximum(y1, b1)
        rbx = jnp.minimum(x2, a2)
        rby = jnp.minimum(y2, b2)
        w = jnp.maximum(rbx - ltx + 1.0, 0.0)
        h = jnp.maximum(rby - lty + 1.0, 0.0)
        inter = w * h
        iou = inter / (area + ai - inter)
        keep = keep_ref[...]
        earlier = (keep > 0.5) & (row < i)
        sup = jnp.any((iou > _NMS_T) & earlier, axis=0, keepdims=True)
        ki = keep_ref[pl.ds(i, 1), :]
        keep_ref[pl.ds(i, 1), :] = jnp.where(sup, 0.0, ki)
        return carry

    lax.fori_loop(1, _K, body, 0)
    outs_ref[...] = jnp.where(keep_ref[...] > 0.5, sc, -1.0)


def _nms(sc_t, creg, cprop):
    return pl.pallas_call(
        _nms_body,
        out_shape=[
            jax.ShapeDtypeStruct((_K, _CF), jnp.float32),
            jax.ShapeDtypeStruct((4, _K, _CF), jnp.float32),
        ],
        scratch_shapes=[
            pltpu.VMEM((_K, _CF), jnp.float32),
            pltpu.VMEM((_K, _CF), jnp.float32),
        ],
    )(sc_t, creg, cprop)


@jax.jit
def kernel(class_logits, box_regression, proposals):
    masked_t = _masked_scores(class_logits)
    top_scores, top_idx = lax.top_k(masked_t, _K)

    cls = jnp.arange(1, _C, dtype=jnp.int32)[:, None]
    rrows = top_idx * _C + cls
    pad_n = _NW * _PW - _CF * _K
    pad_p = (jnp.arange(pad_n, dtype=jnp.int32) * 37) % _N
    rflat = jnp.concatenate([rrows.reshape(-1), pad_p * _C])
    pflat = jnp.concatenate([top_idx.reshape(-1), pad_p])
    ch_off = jnp.arange(4, dtype=jnp.int32)[:, None]
    ridx = (rflat[None, :] * 4 + ch_off).reshape(4, _NW, _PW // 128, 128)
    ridx = ridx.transpose(1, 0, 2, 3)
    pidx = (pflat[None, :] * 4 + ch_off).reshape(4, _NW, _PW // 128, 128)
    pidx = pidx.transpose(1, 0, 2, 3)

    reg_1d = box_regression.reshape(_N * _C * 4)
    prop_1d = proposals.reshape(_N * 4)
    oreg, oprop = _gather_candidates(reg_1d, prop_1d, ridx, pidx)

    creg = oreg.transpose(1, 0, 2).reshape(4, _NW * _PW)[:, : _CF * _K]
    creg = creg.reshape(4, _CF, _K).transpose(0, 2, 1)
    cprop = oprop.transpose(1, 0, 2).reshape(4, _NW * _PW)[:, : _CF * _K]
    cprop = cprop.reshape(4, _CF, _K).transpose(0, 2, 1)
    sc_t = top_scores.T

    outs, outb = _nms(sc_t, creg, cprop)

    flat_scores = outs.T.reshape(-1)
    flat_boxes = outb.transpose(2, 1, 0).reshape(_CF * _K, 4)
    fs, fi = lax.top_k(flat_scores, _DETS)
    top_boxes = flat_boxes[fi]
    top_labels = (fi // _K + 1).astype(jnp.int32)
    return top_boxes, fs, top_labels
